# baseline (speedup 1.0000x reference)
import functools

import jax
import jax.numpy as jnp
from jax.experimental import pallas as pl
from jax.experimental.pallas import tpu as pltpu


def _ffn_body(x_ref, w1g_ref, w1u_ref, w1s_ref, w2_ref, sw2_ref, sg_ref,
              out_ref, a1_ref, a2_ref, *, num_j):
    t = pl.program_id(0)
    dims = (((1,), (1,)), ((), ()))

    dims2 = (((1,), (0,)), ((), ()))
    contrib = jax.lax.dot_general(a1_ref[...], w2_ref[...], dims2,
                                  preferred_element_type=jnp.float32)
    contrib += jax.lax.dot_general(a2_ref[...], sw2_ref[...], dims2,
                                   preferred_element_type=jnp.float32)
    c = t - 1
    init = (jnp.remainder(c, num_j) == 0) | (c < 0)
    prev = jnp.where(init, jnp.float32(0.0), out_ref[...])
    out_ref[...] = prev + contrib

    x = x_ref[...]
    g = jax.lax.dot_general(x, w1g_ref[...], dims,
                            preferred_element_type=jnp.float32)
    u = jax.lax.dot_general(x, w1u_ref[...], dims,
                            preferred_element_type=jnp.float32)
    s = jax.lax.dot_general(x, w1s_ref[...], dims,
                            preferred_element_type=jnp.float32)
    a1_ref[...] = (g * jax.nn.sigmoid(g) * u).astype(jnp.bfloat16)
    a2_ref[...] = (s * jax.nn.sigmoid(s) * sg_ref[...]).astype(jnp.bfloat16)


def kernel(hidden_states, w13, w2, gate, shared_w1, shared_w2, shared_gate_w):
    del gate
    bsz, seq_len, hidden = hidden_states.shape
    n_tokens = bsz * seq_len
    inter = shared_w1.shape[0]

    xf = hidden_states.reshape(n_tokens, hidden)
    x = xf.astype(jnp.bfloat16)
    w13_b = w13.astype(jnp.bfloat16)
    sw1_b = shared_w1.astype(jnp.bfloat16)
    w2t_b = w2.T.astype(jnp.bfloat16)
    sw2t_b = shared_w2.T.astype(jnp.bfloat16)
    sg = jax.nn.sigmoid(xf @ shared_gate_w.T)

    bm = 1024 if n_tokens % 1024 == 0 else n_tokens
    bi = 512 if inter % 512 == 0 else inter
    num_i = n_tokens // bm
    num_j = inter // bi

    def i_first(t):
        return jnp.minimum(t // num_j, num_i - 1)

    def j_first(t):
        return jnp.remainder(t, num_j)

    def i_second(t):
        return jnp.maximum(t - 1, 0) // num_j

    def j_second(t):
        return jnp.remainder(jnp.maximum(t - 1, 0), num_j)

    out = pl.pallas_call(
        functools.partial(_ffn_body, num_j=num_j),
        grid=(num_i * num_j + 1,),
        in_specs=[
            pl.BlockSpec((bm, hidden), lambda t: (i_first(t), 0)),
            pl.BlockSpec((bi, hidden), lambda t: (j_first(t), 0)),
            pl.BlockSpec((bi, hidden),
                         lambda t, nj=num_j: (j_first(t) + nj, 0)),
            pl.BlockSpec((bi, hidden), lambda t: (j_first(t), 0)),
            pl.BlockSpec((bi, hidden), lambda t: (j_second(t), 0)),
            pl.BlockSpec((bi, hidden), lambda t: (j_second(t), 0)),
            pl.BlockSpec((bm, 1), lambda t: (i_first(t), 0)),
        ],
        out_specs=pl.BlockSpec((bm, hidden), lambda t: (i_second(t), 0)),
        out_shape=jax.ShapeDtypeStruct((n_tokens, hidden), jnp.float32),
        scratch_shapes=[pltpu.VMEM((bm, bi), jnp.bfloat16),
                        pltpu.VMEM((bm, bi), jnp.bfloat16)],
        compiler_params=pltpu.CompilerParams(
            dimension_semantics=("arbitrary",),
            vmem_limit_bytes=64 * 1024 * 1024),
    )(x, w13_b, w13_b, sw1_b, w2t_b, sw2t_b, sg)

    return out.reshape(bsz, seq_len, hidden)


# speedup vs baseline: 1.0335x; 1.0335x over previous
import functools

import jax
import jax.numpy as jnp
from jax.experimental import pallas as pl
from jax.experimental.pallas import tpu as pltpu


def _ffn_body(x_ref, w1g_ref, w1u_ref, w1s_ref, w2_ref, sw2_ref, sg_ref,
              out_ref, a1_ref, a2_ref, *, num_j):
    t = pl.program_id(0)
    dims = (((1,), (1,)), ((), ()))

    contrib = jax.lax.dot_general(a1_ref[...], w2_ref[...], dims,
                                  preferred_element_type=jnp.float32)
    contrib += jax.lax.dot_general(a2_ref[...], sw2_ref[...], dims,
                                   preferred_element_type=jnp.float32)
    c = t - 1
    init = (jnp.remainder(c, num_j) == 0) | (c < 0)
    prev = jnp.where(init, jnp.float32(0.0), out_ref[...])
    out_ref[...] = prev + contrib

    x = x_ref[...]
    g = jax.lax.dot_general(x, w1g_ref[...], dims,
                            preferred_element_type=jnp.float32)
    u = jax.lax.dot_general(x, w1u_ref[...], dims,
                            preferred_element_type=jnp.float32)
    s = jax.lax.dot_general(x, w1s_ref[...], dims,
                            preferred_element_type=jnp.float32)
    a1_ref[...] = (g * jax.nn.sigmoid(g) * u).astype(jnp.bfloat16)
    a2_ref[...] = (s * jax.nn.sigmoid(s) * sg_ref[...]).astype(jnp.bfloat16)


def kernel(hidden_states, w13, w2, gate, shared_w1, shared_w2, shared_gate_w):
    del gate
    bsz, seq_len, hidden = hidden_states.shape
    n_tokens = bsz * seq_len
    inter = shared_w1.shape[0]

    xf = hidden_states.reshape(n_tokens, hidden)
    x = xf.astype(jnp.bfloat16)
    w13_b = w13.astype(jnp.bfloat16)
    sw1_b = shared_w1.astype(jnp.bfloat16)
    w2_b = w2.astype(jnp.bfloat16)
    sw2_b = shared_w2.astype(jnp.bfloat16)
    sg = jax.nn.sigmoid(xf @ shared_gate_w.T)

    bm = 1024 if n_tokens % 1024 == 0 else n_tokens
    bi = 512 if inter % 512 == 0 else inter
    num_i = n_tokens // bm
    num_j = inter // bi

    def i_first(t):
        return jnp.minimum(t // num_j, num_i - 1)

    def j_first(t):
        return jnp.remainder(t, num_j)

    def i_second(t):
        return jnp.maximum(t - 1, 0) // num_j

    def j_second(t):
        return jnp.remainder(jnp.maximum(t - 1, 0), num_j)

    out = pl.pallas_call(
        functools.partial(_ffn_body, num_j=num_j),
        grid=(num_i * num_j + 1,),
        in_specs=[
            pl.BlockSpec((bm, hidden), lambda t: (i_first(t), 0)),
            pl.BlockSpec((bi, hidden), lambda t: (j_first(t), 0)),
            pl.BlockSpec((bi, hidden),
                         lambda t, nj=num_j: (j_first(t) + nj, 0)),
            pl.BlockSpec((bi, hidden), lambda t: (j_first(t), 0)),
            pl.BlockSpec((hidden, bi), lambda t: (0, j_second(t))),
            pl.BlockSpec((hidden, bi), lambda t: (0, j_second(t))),
            pl.BlockSpec((bm, 1), lambda t: (i_first(t), 0)),
        ],
        out_specs=pl.BlockSpec((bm, hidden), lambda t: (i_second(t), 0)),
        out_shape=jax.ShapeDtypeStruct((n_tokens, hidden), jnp.float32),
        scratch_shapes=[pltpu.VMEM((bm, bi), jnp.bfloat16),
                        pltpu.VMEM((bm, bi), jnp.bfloat16)],
        compiler_params=pltpu.CompilerParams(
            dimension_semantics=("arbitrary",),
            vmem_limit_bytes=64 * 1024 * 1024),
    )(x, w13_b, w13_b, sw1_b, w2_b, sw2_b, sg)

    return out.reshape(bsz, seq_len, hidden)


# speedup vs baseline: 1.1067x; 1.0708x over previous
import functools

import jax
import jax.numpy as jnp
from jax.experimental import pallas as pl
from jax.experimental.pallas import tpu as pltpu


def _ffn_body(x_ref, w1g_ref, w1u_ref, w1s_ref, w2_ref, sw2_ref, sg_ref,
              out_ref, a1_ref, a2_ref, *, num_j):
    t = pl.program_id(0)
    dims = (((1,), (1,)), ((), ()))

    contrib = jax.lax.dot_general(
        a1_ref[...], w2_ref[...].astype(jnp.bfloat16), dims,
        preferred_element_type=jnp.float32)
    contrib += jax.lax.dot_general(
        a2_ref[...], sw2_ref[...].astype(jnp.bfloat16), dims,
        preferred_element_type=jnp.float32)
    c = t - 1
    init = (jnp.remainder(c, num_j) == 0) | (c < 0)
    prev = jnp.where(init, jnp.float32(0.0), out_ref[...])
    out_ref[...] = prev + contrib

    x = x_ref[...]
    g = jax.lax.dot_general(x, w1g_ref[...], dims,
                            preferred_element_type=jnp.float32)
    u = jax.lax.dot_general(x, w1u_ref[...], dims,
                            preferred_element_type=jnp.float32)
    s = jax.lax.dot_general(x, w1s_ref[...], dims,
                            preferred_element_type=jnp.float32)
    a1_ref[...] = (g * jax.nn.sigmoid(g) * u).astype(jnp.bfloat16)
    a2_ref[...] = (s * jax.nn.sigmoid(s) * sg_ref[...]).astype(jnp.bfloat16)


def kernel(hidden_states, w13, w2, gate, shared_w1, shared_w2, shared_gate_w):
    del gate
    bsz, seq_len, hidden = hidden_states.shape
    n_tokens = bsz * seq_len
    inter = shared_w1.shape[0]

    xf = hidden_states.reshape(n_tokens, hidden)
    x = xf.astype(jnp.bfloat16)
    w13_b = w13.astype(jnp.bfloat16)
    sw1_b = shared_w1.astype(jnp.bfloat16)
    sg = jax.nn.sigmoid(xf @ shared_gate_w.T)

    bm = 1024 if n_tokens % 1024 == 0 else n_tokens
    bi = 512 if inter % 512 == 0 else inter
    num_i = n_tokens // bm
    num_j = inter // bi

    def i_first(t):
        return jnp.minimum(t // num_j, num_i - 1)

    def j_first(t):
        return jnp.remainder(t, num_j)

    def i_second(t):
        return jnp.maximum(t - 1, 0) // num_j

    def j_second(t):
        return jnp.remainder(jnp.maximum(t - 1, 0), num_j)

    out = pl.pallas_call(
        functools.partial(_ffn_body, num_j=num_j),
        grid=(num_i * num_j + 1,),
        in_specs=[
            pl.BlockSpec((bm, hidden), lambda t: (i_first(t), 0)),
            pl.BlockSpec((bi, hidden), lambda t: (j_first(t), 0)),
            pl.BlockSpec((bi, hidden),
                         lambda t, nj=num_j: (j_first(t) + nj, 0)),
            pl.BlockSpec((bi, hidden), lambda t: (j_first(t), 0)),
            pl.BlockSpec((hidden, bi), lambda t: (0, j_second(t))),
            pl.BlockSpec((hidden, bi), lambda t: (0, j_second(t))),
            pl.BlockSpec((bm, 1), lambda t: (i_first(t), 0)),
        ],
        out_specs=pl.BlockSpec((bm, hidden), lambda t: (i_second(t), 0)),
        out_shape=jax.ShapeDtypeStruct((n_tokens, hidden), jnp.float32),
        scratch_shapes=[pltpu.VMEM((bm, bi), jnp.bfloat16),
                        pltpu.VMEM((bm, bi), jnp.bfloat16)],
        compiler_params=pltpu.CompilerParams(
            dimension_semantics=("arbitrary",),
            vmem_limit_bytes=64 * 1024 * 1024),
    )(x, w13_b, w13_b, sw1_b, w2, shared_w2, sg)

    return out.reshape(bsz, seq_len, hidden)


# speedup vs baseline: 1.1270x; 1.0184x over previous
import functools

import jax
import jax.numpy as jnp
from jax.experimental import pallas as pl
from jax.experimental.pallas import tpu as pltpu


def _ffn_body(x_ref, w1g_ref, w1u_ref, w1s_ref, w2_ref, sw2_ref, sg_ref,
              out_ref, a1_ref, a2_ref, *, num_j):
    t = pl.program_id(0)
    dims = (((1,), (1,)), ((), ()))

    contrib = jax.lax.dot_general(
        a1_ref[...], w2_ref[...].astype(jnp.bfloat16), dims,
        preferred_element_type=jnp.float32)
    contrib += jax.lax.dot_general(
        a2_ref[...], sw2_ref[...].astype(jnp.bfloat16), dims,
        preferred_element_type=jnp.float32)
    c = t - 1
    init = (jnp.remainder(c, num_j) == 0) | (c < 0)
    prev = jnp.where(init, jnp.float32(0.0), out_ref[...])
    out_ref[...] = prev + contrib

    x = x_ref[...]
    g = jax.lax.dot_general(x, w1g_ref[...], dims,
                            preferred_element_type=jnp.float32)
    u = jax.lax.dot_general(x, w1u_ref[...], dims,
                            preferred_element_type=jnp.float32)
    s = jax.lax.dot_general(x, w1s_ref[...], dims,
                            preferred_element_type=jnp.float32)
    a1_ref[...] = (g * jax.nn.sigmoid(g) * u).astype(jnp.bfloat16)
    a2_ref[...] = (s * jax.nn.sigmoid(s) * sg_ref[...]).astype(jnp.bfloat16)


def kernel(hidden_states, w13, w2, gate, shared_w1, shared_w2, shared_gate_w):
    del gate
    bsz, seq_len, hidden = hidden_states.shape
    n_tokens = bsz * seq_len
    inter = shared_w1.shape[0]

    x = hidden_states.reshape(n_tokens, hidden).astype(jnp.bfloat16)
    w13_b = w13.astype(jnp.bfloat16)
    sw1_b = shared_w1.astype(jnp.bfloat16)
    sg = jax.nn.sigmoid((x @ shared_gate_w.astype(jnp.bfloat16).T
                         ).astype(jnp.float32))

    bm = 1024 if n_tokens % 1024 == 0 else n_tokens
    bi = 512 if inter % 512 == 0 else inter
    num_i = n_tokens // bm
    num_j = inter // bi

    def i_first(t):
        return jnp.minimum(t // num_j, num_i - 1)

    def j_first(t):
        return jnp.remainder(t, num_j)

    def i_second(t):
        return jnp.maximum(t - 1, 0) // num_j

    def j_second(t):
        return jnp.remainder(jnp.maximum(t - 1, 0), num_j)

    out = pl.pallas_call(
        functools.partial(_ffn_body, num_j=num_j),
        grid=(num_i * num_j + 1,),
        in_specs=[
            pl.BlockSpec((bm, hidden), lambda t: (i_first(t), 0)),
            pl.BlockSpec((bi, hidden), lambda t: (j_first(t), 0)),
            pl.BlockSpec((bi, hidden),
                         lambda t, nj=num_j: (j_first(t) + nj, 0)),
            pl.BlockSpec((bi, hidden), lambda t: (j_first(t), 0)),
            pl.BlockSpec((hidden, bi), lambda t: (0, j_second(t))),
            pl.BlockSpec((hidden, bi), lambda t: (0, j_second(t))),
            pl.BlockSpec((bm, 1), lambda t: (i_first(t), 0)),
        ],
        out_specs=pl.BlockSpec((bm, hidden), lambda t: (i_second(t), 0)),
        out_shape=jax.ShapeDtypeStruct((n_tokens, hidden), jnp.float32),
        scratch_shapes=[pltpu.VMEM((bm, bi), jnp.bfloat16),
                        pltpu.VMEM((bm, bi), jnp.bfloat16)],
        compiler_params=pltpu.CompilerParams(
            dimension_semantics=("arbitrary",),
            vmem_limit_bytes=64 * 1024 * 1024),
    )(x, w13_b, w13_b, sw1_b, w2, shared_w2, sg)

    return out.reshape(bsz, seq_len, hidden)
